# baseline (speedup 1.0000x reference)
import functools

import jax
import jax.numpy as jnp
from jax.experimental import pallas as pl

B, NB, PL, H, F = 16, 64, 24, 1024, 4096
M = B * NB
BM = 128


def _fused_body(vis_ref, lang_ref, invlen_ref, wv_ref, bv_ref, wl_ref, bl_ref,
                lmap_ref, vmap_ref):
    pooled = jnp.sum(lang_ref[...], axis=1) * invlen_ref[...]
    lmap_ref[...] = (
        jax.lax.dot_general(pooled, wl_ref[...], (((1,), (1,)), ((), ())),
                            preferred_element_type=jnp.float32)
        + bl_ref[...]
    )
    vmap_ref[...] = (
        jax.lax.dot_general(vis_ref[...], wv_ref[...], (((1,), (1,)), ((), ())),
                            preferred_element_type=jnp.float32)
        + bv_ref[...]
    )


@functools.partial(jax.jit, static_argnames=())
def kernel(vision, language, phrase_lengths, Wv, bv, Wl, bl):
    vis = vision.reshape(M, F)
    lang = language.reshape(M, PL, H)
    inv_len = (1.0 / phrase_lengths.astype(jnp.float32)).reshape(M, 1)

    grid = (M // BM,)
    lmap, vmap = pl.pallas_call(
        _fused_body,
        grid=grid,
        in_specs=[
            pl.BlockSpec((BM, F), lambda i: (i, 0)),
            pl.BlockSpec((BM, PL, H), lambda i: (i, 0, 0)),
            pl.BlockSpec((BM, 1), lambda i: (i, 0)),
            pl.BlockSpec((H, F), lambda i: (0, 0)),
            pl.BlockSpec((1, H), lambda i: (0, 0)),
            pl.BlockSpec((H, H), lambda i: (0, 0)),
            pl.BlockSpec((1, H), lambda i: (0, 0)),
        ],
        out_specs=[
            pl.BlockSpec((BM, H), lambda i: (i, 0)),
            pl.BlockSpec((BM, H), lambda i: (i, 0)),
        ],
        out_shape=[
            jax.ShapeDtypeStruct((M, H), jnp.float32),
            jax.ShapeDtypeStruct((M, H), jnp.float32),
        ],
    )(vis, lang, inv_len, Wv, bv.reshape(1, H), Wl, bl.reshape(1, H))

    return (lmap.reshape(B, NB, H), vmap.reshape(B, NB, H))
